# baseline (speedup 1.0000x reference)
import functools

import jax
import jax.numpy as jnp
from jax import lax
from jax.experimental import pallas as pl
from jax.experimental.pallas import tpu as pltpu
from jax.experimental.pallas import tpu_sc as plsc

N = 50000
E = 800000
H = 64
L = 3
EPS = 1e-07

BN = 2000
NB = N // BN

_NS = 16
LANE = 128
RPT = 400
E_ROWS = RPT * _NS
E_PAD = E_ROWS * LANE
DESC = 256
DPC = 10
CHE = DESC * DPC
NCH = RPT * LANE // CHE
RING = 3
FPT = 3128
NF = FPT * _NS
NACC = NF


def _ln(x, g, b, eps=1e-5):
    mu = jnp.mean(x, axis=-1, keepdims=True)
    var = jnp.var(x, axis=-1, keepdims=True)
    return (x - mu) * jax.lax.rsqrt(var + eps) * g + b


def _prep_tables(hv, lng, lnb, efcw, efcb, beta):
    hvn = jax.nn.relu(_ln(hv, lng, lnb))
    c = jax.nn.relu(efcw + efcb)
    p = jax.nn.relu(hvn + c) + EPS
    w = jnp.exp(beta * p)
    u = p * w
    return hvn, u, w


def _write_T(T_ref, u, w):
    T_ref[0, :, :] = u[:, 0:32]
    T_ref[1, :, :] = u[:, 32:64]
    T_ref[2, :, :] = w[:, 0:32]
    T_ref[3, :, :] = w[:, 32:64]


def _k0_body(x_ref, nfcw_ref, nfcb_ref, efcw_ref, efcb_ref, lng_ref, lnb_ref,
             beta_ref, hv0_ref, hvn_ref, T_ref):
    hv = jnp.dot(x_ref[...], nfcw_ref[...], preferred_element_type=jnp.float32)
    hv = jax.nn.relu(hv + nfcb_ref[...])
    hv0_ref[...] = hv
    hvn, u, w = _prep_tables(hv, lng_ref[...], lnb_ref[...], efcw_ref[...],
                             efcb_ref[...], beta_ref[...])
    hvn_ref[...] = hvn
    _write_T(T_ref, u, w)


def _k0(x, nfcw, nfcb, efcw, efcb, lng, lnb, beta):
    full = lambda shp: pl.BlockSpec(shp, lambda i: (0,) * len(shp))
    return pl.pallas_call(
        _k0_body,
        grid=(NB,),
        in_specs=[
            pl.BlockSpec((BN, 256), lambda i: (i, 0)),
            full((256, H)), full((1, H)), full((1, H)), full((1, H)),
            full((1, H)), full((1, H)), full((1, 1)),
        ],
        out_specs=[
            pl.BlockSpec((BN, H), lambda i: (i, 0)),
            pl.BlockSpec((BN, H), lambda i: (i, 0)),
            pl.BlockSpec((4, BN, 32), lambda i: (0, i, 0)),
        ],
        out_shape=[
            jax.ShapeDtypeStruct((N, H), jnp.float32),
            jax.ShapeDtypeStruct((N, H), jnp.float32),
            jax.ShapeDtypeStruct((4, N, 32), jnp.float32),
        ],
    )(x, nfcw, nfcb, efcw, efcb, lng, lnb, beta)


def _klayer_body(prep, hvn_ref, acc_ref, hvp_ref, w1_ref, b1_ref, g1_ref,
                 bb1_ref, w2_ref, b2_ref, *rest):
    if prep:
        (efcw_ref, efcb_ref, lng_ref, lnb_ref, beta_ref,
         hv_ref, hvn2_ref, T_ref) = rest
    else:
        (hv_ref,) = rest
    numer = jnp.concatenate([acc_ref[0, :, :], acc_ref[1, :, :]], axis=-1)
    denom = jnp.concatenate([acc_ref[2, :, :], acc_ref[3, :, :]], axis=-1)
    good = denom > 0
    msg = jnp.where(good, numer / jnp.where(good, denom, 1.0), 0.0)
    feats = hvn_ref[...] + msg
    h = jnp.dot(feats, w1_ref[...], preferred_element_type=jnp.float32)
    h = jax.nn.relu(_ln(h + b1_ref[...], g1_ref[...], bb1_ref[...]))
    hv = jnp.dot(h, w2_ref[...], preferred_element_type=jnp.float32)
    hv = hv + b2_ref[...] + hvp_ref[...]
    hv_ref[...] = hv
    if prep:
        hvn, u, w = _prep_tables(hv, lng_ref[...], lnb_ref[...], efcw_ref[...],
                                 efcb_ref[...], beta_ref[...])
        hvn2_ref[...] = hvn
        _write_T(T_ref, u, w)


def _klayer(prep, hvn, acc, hvp, w1, b1, g1, bb1, w2, b2, *extra):
    full = lambda shp: pl.BlockSpec(shp, lambda i: (0,) * len(shp))
    rowspec = pl.BlockSpec((BN, H), lambda i: (i, 0))
    in_specs = [
        rowspec,
        pl.BlockSpec((4, BN, 32), lambda i: (0, i, 0)),
        rowspec,
        full((H, 2 * H)), full((1, 2 * H)), full((1, 2 * H)),
        full((1, 2 * H)), full((2 * H, H)), full((1, H)),
    ]
    out_specs = [rowspec]
    out_shape = [jax.ShapeDtypeStruct((N, H), jnp.float32)]
    if prep:
        in_specs += [full((1, H)), full((1, H)), full((1, H)), full((1, H)),
                     full((1, 1))]
        out_specs += [rowspec, pl.BlockSpec((4, BN, 32), lambda i: (0, i, 0))]
        out_shape += [jax.ShapeDtypeStruct((N, H), jnp.float32),
                      jax.ShapeDtypeStruct((4, N, 32), jnp.float32)]
    return pl.pallas_call(
        functools.partial(_klayer_body, prep),
        grid=(NB,),
        in_specs=in_specs,
        out_specs=out_specs,
        out_shape=out_shape,
    )(hvn, acc, hvp, w1, b1, g1, bb1, w2, b2, *extra)


def _k4_body(hv0_ref, hv1_ref, hv2_ref, hv3_ref, phiw_ref, phib_ref, wa_ref,
             ba_ref, wb_ref, bbat_ref, wc_ref, bc_ref, rhow_ref, rhob_ref,
             clsw_ref, clsb_ref, hpath_ref, out_ref, outfeat_ref,
             acch_ref, acce_ref):
    i = pl.program_id(0)
    xcat = jnp.concatenate(
        [hv0_ref[...], hv1_ref[...], hv2_ref[...], hv3_ref[...]], axis=-1)
    hp = jnp.dot(xcat, phiw_ref[...], preferred_element_type=jnp.float32)
    hp = jax.nn.relu(hp + phib_ref[...])
    hpath_ref[...] = hp
    a = jnp.tanh(jnp.dot(hp, wa_ref[...], preferred_element_type=jnp.float32)
                 + ba_ref[...])
    b = jax.nn.sigmoid(
        jnp.dot(hp, wb_ref[...], preferred_element_type=jnp.float32)
        + bbat_ref[...])
    gate = jnp.dot(a * b, wc_ref[...], preferred_element_type=jnp.float32)
    gate = gate + bc_ref[...]
    shift = jnp.sum(jnp.abs(wc_ref[...])) + jnp.abs(bc_ref[0, 0])
    e = jnp.exp(gate - shift)
    se = jnp.sum(e)
    seh = jnp.sum(e * hp, axis=0, keepdims=True)

    @pl.when(i == 0)
    def _():
        acch_ref[...] = seh
        acce_ref[...] = jnp.full((1, 256), se, jnp.float32)

    @pl.when(i > 0)
    def _():
        acch_ref[...] += seh
        acce_ref[...] += jnp.full((1, 256), se, jnp.float32)

    @pl.when(i == NB - 1)
    def _():
        hg = acch_ref[...] / acce_ref[...]
        of = jnp.dot(hg, rhow_ref[...], preferred_element_type=jnp.float32)
        of = jax.nn.relu(of + rhob_ref[...])
        outfeat_ref[...] = of
        out_ref[...] = jnp.dot(of, clsw_ref[...],
                               preferred_element_type=jnp.float32) + clsb_ref[...]


def _k4(hv0, hv1, hv2, hv3, phiw, phib, wa, ba, wb, bbat, wc, bc, rhow, rhob,
        clsw, clsb):
    full = lambda shp: pl.BlockSpec(shp, lambda i: (0,) * len(shp))
    rowspec = pl.BlockSpec((BN, H), lambda i: (i, 0))
    D = 4 * H
    return pl.pallas_call(
        _k4_body,
        grid=(NB,),
        in_specs=[
            rowspec, rowspec, rowspec, rowspec,
            full((D, D)), full((1, D)), full((D, D)), full((1, D)),
            full((D, D)), full((1, D)), full((D, 1)), full((1, 1)),
            full((D, H)), full((1, H)), full((H, 2)), full((1, 2)),
        ],
        out_specs=[
            pl.BlockSpec((BN, D), lambda i: (i, 0)),
            full((1, 2)), full((1, H)),
        ],
        out_shape=[
            jax.ShapeDtypeStruct((N, D), jnp.float32),
            jax.ShapeDtypeStruct((1, 2), jnp.float32),
            jax.ShapeDtypeStruct((1, H), jnp.float32),
        ],
        scratch_shapes=[
            pltpu.VMEM((1, D), jnp.float32),
            pltpu.VMEM((1, D), jnp.float32),
        ],
    )(hv0, hv1, hv2, hv3, phiw, phib, wa, ba, wb, bbat, wc, bc, rhow, rhob,
      clsw, clsb)


def _sc_body(T_ref, src_ref, dst_ref, zr_ref, out_ref,
             sacc, isb, idb, rbufs, gsems, ssems):
    c = lax.axis_index("c")
    s = lax.axis_index("s")
    fl0 = s * FPT
    r_base = s * RPT

    def zero_own_range():
        pltpu.sync_copy(zr_ref, rbufs[0])

        def zstep(j, _):
            pltpu.sync_copy(rbufs[0], sacc.at[pl.ds(fl0 + j * DESC, DESC)])
            return 0
        lax.fori_loop(0, FPT // DESC, zstep, 0)
        pltpu.sync_copy(rbufs[0].at[pl.ds(0, FPT - (FPT // DESC) * DESC)],
                        sacc.at[pl.ds(fl0 + (FPT // DESC) * DESC,
                                      FPT - (FPT // DESC) * DESC)])

    def fire_g(d, k):
        pltpu.async_copy(T_ref.at[isb.at[pl.ds(d * DESC, DESC)]],
                         rbufs[k], gsems[k])

    def drain_g(d, k):
        pltpu.make_async_copy(T_ref.at[isb.at[pl.ds(d * DESC, DESC)]],
                              rbufs[k], gsems[k]).wait()

    def fire_s(d, k):
        pltpu.async_copy(rbufs[k], sacc.at[idb.at[pl.ds(d * DESC, DESC)]],
                         ssems[k], add=True)

    def drain_s(d, k):
        pltpu.make_async_copy(rbufs[k],
                              sacc.at[idb.at[pl.ds(d * DESC, DESC)]],
                              ssems[k]).wait()

    for sl in range(2):
        slab = 2 * c + sl
        zero_own_range()
        plsc.subcore_barrier()

        def chunk(ci, _):
            e0 = (r_base + ci * (CHE // LANE)) * LANE
            pltpu.sync_copy(src_ref.at[slab, pl.ds(e0, CHE)], isb)
            pltpu.sync_copy(dst_ref.at[pl.ds(e0, CHE)], idb)
            fire_g(0, 0)
            fire_g(1, 1)
            for d in range(DPC):
                k = d % RING
                kn = (d + 2) % RING
                if 1 <= d <= DPC - 3:
                    drain_s(d - 1, kn)
                if d <= DPC - 3:
                    fire_g(d + 2, kn)
                drain_g(d, k)
                fire_s(d, k)
            for d in range(DPC - 3, DPC):
                drain_s(d, d % RING)
            return 0
        lax.fori_loop(0, NCH, chunk, 0)

        plsc.subcore_barrier()
        pltpu.sync_copy(sacc.at[pl.ds(fl0, FPT)],
                        out_ref.at[pl.ds(slab * NF + fl0, FPT)])


@functools.partial(
    pl.kernel,
    out_type=jax.ShapeDtypeStruct((4 * NF, 32), jnp.float32),
    mesh=plsc.VectorSubcoreMesh(core_axis_name="c", subcore_axis_name="s"),
    scratch_types=(
        [pltpu.VMEM_SHARED((NACC, 32), jnp.float32),
         pltpu.VMEM((CHE,), jnp.int32),
         pltpu.VMEM((CHE,), jnp.int32)]
        + [pltpu.VMEM((DESC, 32), jnp.float32)] * RING
        + [pltpu.SemaphoreType.DMA] * (2 * RING)
    ),
    compiler_params=pltpu.CompilerParams(use_tc_tiling_on_sc=False),
)
def _sc_scatter(T_ref, src_ref, dst_ref, zr_ref, out_ref,
                sacc, isb, idb, *rest):
    rbufs = rest[:RING]
    gsems = rest[RING:2 * RING]
    ssems = rest[2 * RING:3 * RING]
    _sc_body(T_ref, src_ref, dst_ref, zr_ref, out_ref,
             sacc, isb, idb, rbufs, gsems, ssems)


def kernel(x, edge_index, nfc_w, nfc_b, efc_w, efc_b, ln_g, ln_b, betas,
           w1, b1, g1, bb1, w2, b2, phi_w, phi_b, wa, ba, wb, bb_attn,
           wc, bc, rho_w, rho_b, cls_w, cls_b):
    f32 = jnp.float32
    src = edge_index[0].astype(jnp.int32)
    dst = edge_index[1].astype(jnp.int32)
    pad = E_PAD - E
    srcp = jnp.concatenate([src, jnp.zeros((pad,), jnp.int32)])
    dstp = jnp.concatenate([dst, jnp.full((pad,), N, jnp.int32)])
    src4 = jnp.stack([srcp + s * N for s in range(4)])
    dst2 = dstp
    zrows = jnp.zeros((DESC, 32), f32)

    r2 = lambda v: v.reshape(1, -1)
    hv0, hvn, T = _k0(x, nfc_w, r2(nfc_b), efc_w, r2(efc_b),
                      r2(ln_g[0]), r2(ln_b[0]), betas[0].reshape(1, 1))
    hvs = [hv0]
    for l in range(L):
        acc = _sc_scatter(T.reshape(4 * N, 32), src4, dst2, zrows)
        acc = acc.reshape(4, NF, 32)
        prep = l < L - 1
        extra = ()
        if prep:
            extra = (efc_w, r2(efc_b), r2(ln_g[l + 1]), r2(ln_b[l + 1]),
                     betas[l + 1].reshape(1, 1))
        res = _klayer(prep, hvn, acc, hvs[-1], w1[l], r2(b1[l]), r2(g1[l]),
                      r2(bb1[l]), w2[l], r2(b2[l]), *extra)
        if prep:
            hv, hvn, T = res
        else:
            (hv,) = res
        hvs.append(hv)

    h_path, out, out_feat = _k4(hvs[0], hvs[1], hvs[2], hvs[3], phi_w,
                                r2(phi_b), wa, r2(ba), wb, r2(bb_attn), wc,
                                bc.reshape(1, 1), rho_w, r2(rho_b), cls_w,
                                r2(cls_b))
    return (out, out_feat, h_path)


# speedup vs baseline: 1.7037x; 1.7037x over previous
import functools

import jax
import jax.numpy as jnp
from jax import lax
from jax.experimental import pallas as pl
from jax.experimental.pallas import tpu as pltpu
from jax.experimental.pallas import tpu_sc as plsc

N = 50000
E = 800000
H = 64
L = 3
EPS = 1e-07

BN = 2000
NB = N // BN

_NS = 16
LANE = 128
RPT = 400
E_ROWS = RPT * _NS
E_PAD = E_ROWS * LANE
DESC = 128
DPC = 40
CHE = DESC * DPC
NCH = RPT * LANE // CHE
RING = 5
FPT = 3128
NF = FPT * _NS


def _ln(x, g, b, eps=1e-5):
    mu = jnp.mean(x, axis=-1, keepdims=True)
    var = jnp.var(x, axis=-1, keepdims=True)
    return (x - mu) * jax.lax.rsqrt(var + eps) * g + b


def _prep_tables(hv, lng, lnb, efcw, efcb, beta):
    hvn = jax.nn.relu(_ln(hv, lng, lnb))
    c = jax.nn.relu(efcw + efcb)
    p = jax.nn.relu(hvn + c) + EPS
    w = jnp.exp(beta * p)
    u = p * w
    return hvn, u, w


def _write_T(T_ref, u, w):
    T_ref[0, :, :] = u.astype(jnp.bfloat16)
    T_ref[1, :, :] = w.astype(jnp.bfloat16)


def _k0_body(x_ref, nfcw_ref, nfcb_ref, efcw_ref, efcb_ref, lng_ref, lnb_ref,
             beta_ref, hv0_ref, hvn_ref, T_ref):
    hv = jnp.dot(x_ref[...], nfcw_ref[...], preferred_element_type=jnp.float32)
    hv = jax.nn.relu(hv + nfcb_ref[...])
    hv0_ref[...] = hv
    hvn, u, w = _prep_tables(hv, lng_ref[...], lnb_ref[...], efcw_ref[...],
                             efcb_ref[...], beta_ref[...])
    hvn_ref[...] = hvn
    _write_T(T_ref, u, w)


def _k0(x, nfcw, nfcb, efcw, efcb, lng, lnb, beta):
    full = lambda shp: pl.BlockSpec(shp, lambda i: (0,) * len(shp))
    return pl.pallas_call(
        _k0_body,
        grid=(NB,),
        in_specs=[
            pl.BlockSpec((BN, 256), lambda i: (i, 0)),
            full((256, H)), full((1, H)), full((1, H)), full((1, H)),
            full((1, H)), full((1, H)), full((1, 1)),
        ],
        out_specs=[
            pl.BlockSpec((BN, H), lambda i: (i, 0)),
            pl.BlockSpec((BN, H), lambda i: (i, 0)),
            pl.BlockSpec((2, BN, H), lambda i: (0, i, 0)),
        ],
        out_shape=[
            jax.ShapeDtypeStruct((N, H), jnp.float32),
            jax.ShapeDtypeStruct((N, H), jnp.float32),
            jax.ShapeDtypeStruct((2, N, H), jnp.bfloat16),
        ],
    )(x, nfcw, nfcb, efcw, efcb, lng, lnb, beta)


def _klayer_body(prep, hvn_ref, acc_ref, hvp_ref, w1_ref, b1_ref, g1_ref,
                 bb1_ref, w2_ref, b2_ref, *rest):
    if prep:
        (efcw_ref, efcb_ref, lng_ref, lnb_ref, beta_ref,
         hv_ref, hvn2_ref, T_ref) = rest
    else:
        (hv_ref,) = rest
    numer = acc_ref[0, :, :].astype(jnp.float32)
    denom = acc_ref[1, :, :].astype(jnp.float32)
    good = denom > 0
    msg = jnp.where(good, numer / jnp.where(good, denom, 1.0), 0.0)
    feats = hvn_ref[...] + msg
    h = jnp.dot(feats, w1_ref[...], preferred_element_type=jnp.float32)
    h = jax.nn.relu(_ln(h + b1_ref[...], g1_ref[...], bb1_ref[...]))
    hv = jnp.dot(h, w2_ref[...], preferred_element_type=jnp.float32)
    hv = hv + b2_ref[...] + hvp_ref[...]
    hv_ref[...] = hv
    if prep:
        hvn, u, w = _prep_tables(hv, lng_ref[...], lnb_ref[...], efcw_ref[...],
                                 efcb_ref[...], beta_ref[...])
        hvn2_ref[...] = hvn
        _write_T(T_ref, u, w)


def _klayer(prep, hvn, acc, hvp, w1, b1, g1, bb1, w2, b2, *extra):
    full = lambda shp: pl.BlockSpec(shp, lambda i: (0,) * len(shp))
    rowspec = pl.BlockSpec((BN, H), lambda i: (i, 0))
    in_specs = [
        rowspec,
        pl.BlockSpec((2, BN, H), lambda i: (0, i, 0)),
        rowspec,
        full((H, 2 * H)), full((1, 2 * H)), full((1, 2 * H)),
        full((1, 2 * H)), full((2 * H, H)), full((1, H)),
    ]
    out_specs = [rowspec]
    out_shape = [jax.ShapeDtypeStruct((N, H), jnp.float32)]
    if prep:
        in_specs += [full((1, H)), full((1, H)), full((1, H)), full((1, H)),
                     full((1, 1))]
        out_specs += [rowspec, pl.BlockSpec((2, BN, H), lambda i: (0, i, 0))]
        out_shape += [jax.ShapeDtypeStruct((N, H), jnp.float32),
                      jax.ShapeDtypeStruct((2, N, H), jnp.bfloat16)]
    return pl.pallas_call(
        functools.partial(_klayer_body, prep),
        grid=(NB,),
        in_specs=in_specs,
        out_specs=out_specs,
        out_shape=out_shape,
    )(hvn, acc, hvp, w1, b1, g1, bb1, w2, b2, *extra)


def _k4_body(hv0_ref, hv1_ref, hv2_ref, hv3_ref, phiw_ref, phib_ref, wa_ref,
             ba_ref, wb_ref, bbat_ref, wc_ref, bc_ref, rhow_ref, rhob_ref,
             clsw_ref, clsb_ref, hpath_ref, out_ref, outfeat_ref,
             acch_ref, acce_ref):
    i = pl.program_id(0)
    xcat = jnp.concatenate(
        [hv0_ref[...], hv1_ref[...], hv2_ref[...], hv3_ref[...]], axis=-1)
    hp = jnp.dot(xcat, phiw_ref[...], preferred_element_type=jnp.float32)
    hp = jax.nn.relu(hp + phib_ref[...])
    hpath_ref[...] = hp
    a = jnp.tanh(jnp.dot(hp, wa_ref[...], preferred_element_type=jnp.float32)
                 + ba_ref[...])
    b = jax.nn.sigmoid(
        jnp.dot(hp, wb_ref[...], preferred_element_type=jnp.float32)
        + bbat_ref[...])
    gate = jnp.dot(a * b, wc_ref[...], preferred_element_type=jnp.float32)
    gate = gate + bc_ref[...]
    shift = jnp.sum(jnp.abs(wc_ref[...])) + jnp.abs(bc_ref[0, 0])
    e = jnp.exp(gate - shift)
    se = jnp.sum(e)
    seh = jnp.sum(e * hp, axis=0, keepdims=True)

    @pl.when(i == 0)
    def _():
        acch_ref[...] = seh
        acce_ref[...] = jnp.full((1, 256), se, jnp.float32)

    @pl.when(i > 0)
    def _():
        acch_ref[...] += seh
        acce_ref[...] += jnp.full((1, 256), se, jnp.float32)

    @pl.when(i == NB - 1)
    def _():
        hg = acch_ref[...] / acce_ref[...]
        of = jnp.dot(hg, rhow_ref[...], preferred_element_type=jnp.float32)
        of = jax.nn.relu(of + rhob_ref[...])
        outfeat_ref[...] = of
        out_ref[...] = jnp.dot(of, clsw_ref[...],
                               preferred_element_type=jnp.float32) + clsb_ref[...]


def _k4(hv0, hv1, hv2, hv3, phiw, phib, wa, ba, wb, bbat, wc, bc, rhow, rhob,
        clsw, clsb):
    full = lambda shp: pl.BlockSpec(shp, lambda i: (0,) * len(shp))
    rowspec = pl.BlockSpec((BN, H), lambda i: (i, 0))
    D = 4 * H
    return pl.pallas_call(
        _k4_body,
        grid=(NB,),
        in_specs=[
            rowspec, rowspec, rowspec, rowspec,
            full((D, D)), full((1, D)), full((D, D)), full((1, D)),
            full((D, D)), full((1, D)), full((D, 1)), full((1, 1)),
            full((D, H)), full((1, H)), full((H, 2)), full((1, 2)),
        ],
        out_specs=[
            pl.BlockSpec((BN, D), lambda i: (i, 0)),
            full((1, 2)), full((1, H)),
        ],
        out_shape=[
            jax.ShapeDtypeStruct((N, D), jnp.float32),
            jax.ShapeDtypeStruct((1, 2), jnp.float32),
            jax.ShapeDtypeStruct((1, H), jnp.float32),
        ],
        scratch_shapes=[
            pltpu.VMEM((1, D), jnp.float32),
            pltpu.VMEM((1, D), jnp.float32),
        ],
    )(hv0, hv1, hv2, hv3, phiw, phib, wa, ba, wb, bbat, wc, bc, rhow, rhob,
      clsw, clsb)


def _sc_body(T_ref, src_ref, dst_ref, zr_ref, out_ref,
             sacc, isb, idb, rbufs, gsems, ssems):
    c = lax.axis_index("c")
    s = lax.axis_index("s")
    fl0 = s * FPT
    e_base = s * RPT * LANE

    pltpu.sync_copy(zr_ref, rbufs[0])

    def zstep(j, _):
        pltpu.sync_copy(rbufs[0], sacc.at[pl.ds(fl0 + j * DESC, DESC)])
        return 0
    lax.fori_loop(0, FPT // DESC, zstep, 0)
    pltpu.sync_copy(rbufs[0].at[pl.ds(0, FPT - (FPT // DESC) * DESC)],
                    sacc.at[pl.ds(fl0 + (FPT // DESC) * DESC,
                                  FPT - (FPT // DESC) * DESC)])
    plsc.subcore_barrier()

    def fire_g(d, k):
        pltpu.async_copy(T_ref.at[isb.at[pl.ds(d * DESC, DESC)]],
                         rbufs[k], gsems[k])

    def drain_g(d, k):
        pltpu.make_async_copy(T_ref.at[isb.at[pl.ds(d * DESC, DESC)]],
                              rbufs[k], gsems[k]).wait()

    def fire_s(d, k):
        pltpu.async_copy(rbufs[k], sacc.at[idb.at[pl.ds(d * DESC, DESC)]],
                         ssems[k], add=True)

    def drain_s(d, k):
        pltpu.make_async_copy(rbufs[k],
                              sacc.at[idb.at[pl.ds(d * DESC, DESC)]],
                              ssems[k]).wait()

    def chunk(ci, _):
        e0 = e_base + ci * CHE
        pltpu.sync_copy(src_ref.at[c, pl.ds(e0, CHE)], isb)
        pltpu.sync_copy(dst_ref.at[pl.ds(e0, CHE)], idb)
        for t in range(RING - 1):
            fire_g(t, t)

        def group(g, _):
            for k in range(RING):
                d = g * RING + k
                kn = (k + RING - 1) % RING

                @pl.when(d + RING - 1 < DPC)
                def _():
                    @pl.when(d >= 1)
                    def _():
                        drain_s(d - 1, kn)
                    fire_g(d + RING - 1, kn)
                drain_g(d, k)
                fire_s(d, k)
            return 0
        lax.fori_loop(0, DPC // RING, group, 0)
        for d in range(DPC - RING, DPC):
            drain_s(d, d % RING)
        return 0
    lax.fori_loop(0, NCH, chunk, 0)

    plsc.subcore_barrier()
    pltpu.sync_copy(sacc.at[pl.ds(fl0, FPT)],
                    out_ref.at[pl.ds(c * NF + fl0, FPT)])


@functools.partial(
    pl.kernel,
    out_type=jax.ShapeDtypeStruct((2 * NF, H), jnp.bfloat16),
    mesh=plsc.VectorSubcoreMesh(core_axis_name="c", subcore_axis_name="s"),
    scratch_types=(
        [pltpu.VMEM_SHARED((NF, H), jnp.bfloat16),
         pltpu.VMEM((CHE,), jnp.int32),
         pltpu.VMEM((CHE,), jnp.int32)]
        + [pltpu.VMEM((DESC, H), jnp.bfloat16)] * RING
        + [pltpu.SemaphoreType.DMA] * (2 * RING)
    ),
    compiler_params=pltpu.CompilerParams(use_tc_tiling_on_sc=False),
)
def _sc_scatter(T_ref, src_ref, dst_ref, zr_ref, out_ref,
                sacc, isb, idb, *rest):
    rbufs = rest[:RING]
    gsems = rest[RING:2 * RING]
    ssems = rest[2 * RING:3 * RING]
    _sc_body(T_ref, src_ref, dst_ref, zr_ref, out_ref,
             sacc, isb, idb, rbufs, gsems, ssems)


def kernel(x, edge_index, nfc_w, nfc_b, efc_w, efc_b, ln_g, ln_b, betas,
           w1, b1, g1, bb1, w2, b2, phi_w, phi_b, wa, ba, wb, bb_attn,
           wc, bc, rho_w, rho_b, cls_w, cls_b):
    f32 = jnp.float32
    src = edge_index[0].astype(jnp.int32)
    dst = edge_index[1].astype(jnp.int32)
    pad = E_PAD - E
    srcp = jnp.concatenate([src, jnp.zeros((pad,), jnp.int32)])
    dstp = jnp.concatenate([dst, jnp.full((pad,), N, jnp.int32)])
    src2 = jnp.stack([srcp, srcp + N])
    zrows = jnp.zeros((DESC, H), jnp.bfloat16)

    r2 = lambda v: v.reshape(1, -1)
    hv0, hvn, T = _k0(x, nfc_w, r2(nfc_b), efc_w, r2(efc_b),
                      r2(ln_g[0]), r2(ln_b[0]), betas[0].reshape(1, 1))
    hvs = [hv0]
    for l in range(L):
        acc = _sc_scatter(T.reshape(2 * N, H), src2, dstp, zrows)
        acc = acc.reshape(2, NF, H)
        prep = l < L - 1
        extra = ()
        if prep:
            extra = (efc_w, r2(efc_b), r2(ln_g[l + 1]), r2(ln_b[l + 1]),
                     betas[l + 1].reshape(1, 1))
        res = _klayer(prep, hvn, acc, hvs[-1], w1[l], r2(b1[l]), r2(g1[l]),
                      r2(bb1[l]), w2[l], r2(b2[l]), *extra)
        if prep:
            hv, hvn, T = res
        else:
            (hv,) = res
        hvs.append(hv)

    h_path, out, out_feat = _k4(hvs[0], hvs[1], hvs[2], hvs[3], phi_w,
                                r2(phi_b), wa, r2(ba), wb, r2(bb_attn), wc,
                                bc.reshape(1, 1), rho_w, r2(rho_b), cls_w,
                                r2(cls_b))
    return (out, out_feat, h_path)
